# baseline (speedup 1.0000x reference)
import functools

import jax
import jax.numpy as jnp
from jax import lax
from jax.experimental import pallas as pl
from jax.experimental.pallas import tpu as pltpu
from jax.experimental.pallas import tpu_sc as plsc

NC = 2
NS = 16
NW = NC * NS
C = 128


def _deg_kernel(n_pad, kc):
    zb = n_pad // NS
    mesh = plsc.VectorSubcoreMesh(core_axis_name="c", subcore_axis_name="s")

    @functools.partial(
        pl.kernel,
        out_type=jax.ShapeDtypeStruct((NC, n_pad), jnp.float32),
        mesh=mesh,
        scratch_types=[
            pltpu.VMEM((kc, C), jnp.int32),
            pltpu.VMEM((C,), jnp.float32),
            pltpu.VMEM((zb,), jnp.float32),
            pltpu.VMEM_SHARED((n_pad,), jnp.float32),
        ],
    )
    def deg_kernel(dst_hbm, degp_hbm, dst_v, ones_v, zeros_v, deg_sh):
        c = lax.axis_index("c")
        s = lax.axis_index("s")
        wid = c * NS + s
        pltpu.sync_copy(dst_hbm.at[wid], dst_v)

        for i in range(C // 16):
            ones_v[pl.ds(i * 16, 16)] = jnp.ones((16,), jnp.float32)

        def zfill(j, _):
            zeros_v[pl.ds(j * 16, 16)] = jnp.zeros((16,), jnp.float32)
            return 0

        lax.fori_loop(0, zb // 16, zfill, 0)
        pltpu.sync_copy(zeros_v, deg_sh.at[pl.ds(s * zb, zb)])
        plsc.subcore_barrier()

        def step(j, _):
            pltpu.sync_copy(ones_v, deg_sh.at[dst_v.at[j]], add=True)
            return 0

        lax.fori_loop(0, kc, step, 0)
        plsc.subcore_barrier()
        pltpu.sync_copy(deg_sh.at[pl.ds(s * zb, zb)],
                        degp_hbm.at[c, pl.ds(s * zb, zb)])

    return deg_kernel


def _scatter_kernel(n_pad, d, kc):
    zrows = n_pad // NS
    zc = 80
    mesh = plsc.VectorSubcoreMesh(core_axis_name="c", subcore_axis_name="s")

    @functools.partial(
        pl.kernel,
        out_type=jax.ShapeDtypeStruct((NC, n_pad, d), jnp.float32),
        mesh=mesh,
        compiler_params=pltpu.CompilerParams(use_tc_tiling_on_sc=False),
        scratch_types=[
            pltpu.VMEM((kc, C), jnp.int32),
            pltpu.VMEM((kc, C), jnp.int32),
            pltpu.VMEM((C, d), jnp.float32),
            pltpu.VMEM((zc, d), jnp.float32),
            pltpu.VMEM_SHARED((n_pad, d), jnp.float32),
            pltpu.SemaphoreType.DMA,
        ],
    )
    def scatter_kernel(h2_hbm, src_hbm, dst_hbm, outp_hbm,
                       src_v, dst_v, rows0, zrows_v, out_sh, gs0):
        bufs = [rows0]
        gs = [gs0]
        c = lax.axis_index("c")
        s = lax.axis_index("s")
        wid = c * NS + s
        pltpu.sync_copy(src_hbm.at[wid], src_v)
        pltpu.sync_copy(dst_hbm.at[wid], dst_v)

        def zfill(r, _):
            for k in range(d // 16):
                zrows_v[r, pl.ds(k * 16, 16)] = jnp.zeros((16,), jnp.float32)
            return 0

        lax.fori_loop(0, zc, zfill, 0)
        base = s * zrows
        for t in range(zrows // zc):
            pltpu.sync_copy(zrows_v, out_sh.at[pl.ds(base + t * zc, zc)])
        plsc.subcore_barrier()

        def step(j, _):
            pltpu.async_copy(h2_hbm.at[src_v.at[j]], bufs[0], gs[0]).wait()
            pltpu.sync_copy(bufs[0], out_sh.at[dst_v.at[j]], add=True)
            return 0

        lax.fori_loop(0, kc, step, 0)

        plsc.subcore_barrier()
        pltpu.sync_copy(out_sh.at[pl.ds(base, zrows)],
                        outp_hbm.at[c, pl.ds(base, zrows)])

    return scatter_kernel


def _h2_body(x_ref, w_ref, b_ref, degp_ref, h2_ref, dinv_ref):
    deg = jnp.sum(degp_ref[...], axis=1, keepdims=True) + 1.0
    dinv = lax.rsqrt(deg)
    h = jnp.dot(x_ref[...], w_ref[...], preferred_element_type=jnp.float32)
    h2_ref[...] = (h + b_ref[...]) * dinv
    dinv_ref[...] = dinv


def _final_body(outp_ref, h2_ref, dinv_ref, out_ref):
    acc = outp_ref[0, :, :]
    for i in range(1, NC):
        acc = acc + outp_ref[i, :, :]
    dinv = dinv_ref[...]
    out_ref[...] = jnp.maximum(dinv * (acc + h2_ref[...]), 0.0)


def kernel(x, edge_index, W, b):
    n, d = x.shape
    e = edge_index.shape[1]

    n_pad = ((n + NS * 16 - 1) // (NS * 16)) * (NS * 16)
    blk = 4 * NW * C
    e_pad = ((e + blk - 1) // blk) * blk
    kc = e_pad // (NW * C)

    src = edge_index[0]
    dst = edge_index[1]
    if e_pad != e:
        src = jnp.concatenate([src, jnp.zeros((e_pad - e,), jnp.int32)])
        dst = jnp.concatenate(
            [dst, jnp.full((e_pad - e,), n_pad - 1, jnp.int32)])
    src_r = src.reshape(NW, kc, C)
    dst_r = dst.reshape(NW, kc, C)

    x_pad = jnp.pad(x, ((0, n_pad - n), (0, 0))) if n_pad != n else x

    degp = _deg_kernel(n_pad, kc)(dst_r)
    degp_t = degp.T

    h2, dinv = pl.pallas_call(
        _h2_body,
        out_shape=[
            jax.ShapeDtypeStruct((n_pad, d), jnp.float32),
            jax.ShapeDtypeStruct((n_pad, 1), jnp.float32),
        ],
    )(x_pad, W, b.reshape(1, d), degp_t)

    outp = _scatter_kernel(n_pad, d, kc)(h2, src_r, dst_r)

    out = pl.pallas_call(
        _final_body,
        out_shape=jax.ShapeDtypeStruct((n_pad, d), jnp.float32),
    )(outp, h2, dinv)
    return out[:n]


# speedup vs baseline: 2.3202x; 2.3202x over previous
import functools

import jax
import jax.numpy as jnp
from jax import lax
from jax.experimental import pallas as pl
from jax.experimental.pallas import tpu as pltpu
from jax.experimental.pallas import tpu_sc as plsc

NC = 2
NS = 16
NW = NC * NS
C = 128


def _deg_kernel(n_pad, kc):
    zb = n_pad // NS
    mesh = plsc.VectorSubcoreMesh(core_axis_name="c", subcore_axis_name="s")

    @functools.partial(
        pl.kernel,
        out_type=jax.ShapeDtypeStruct((NC, n_pad), jnp.float32),
        mesh=mesh,
        scratch_types=[
            pltpu.VMEM((kc, C), jnp.int32),
            pltpu.VMEM((C,), jnp.float32),
            pltpu.VMEM((zb,), jnp.float32),
            pltpu.VMEM_SHARED((n_pad,), jnp.float32),
        ],
    )
    def deg_kernel(dst_hbm, degp_hbm, dst_v, ones_v, zeros_v, deg_sh):
        c = lax.axis_index("c")
        s = lax.axis_index("s")
        wid = c * NS + s
        pltpu.sync_copy(dst_hbm.at[wid], dst_v)

        for i in range(C // 16):
            ones_v[pl.ds(i * 16, 16)] = jnp.ones((16,), jnp.float32)

        def zfill(j, _):
            zeros_v[pl.ds(j * 16, 16)] = jnp.zeros((16,), jnp.float32)
            return 0

        lax.fori_loop(0, zb // 16, zfill, 0)
        pltpu.sync_copy(zeros_v, deg_sh.at[pl.ds(s * zb, zb)])
        plsc.subcore_barrier()

        def step(j, _):
            pltpu.sync_copy(ones_v, deg_sh.at[dst_v.at[j]], add=True)
            return 0

        lax.fori_loop(0, kc, step, 0)
        plsc.subcore_barrier()
        pltpu.sync_copy(deg_sh.at[pl.ds(s * zb, zb)],
                        degp_hbm.at[c, pl.ds(s * zb, zb)])

    return deg_kernel


def _scatter_kernel(n_pad, dh, kc):
    zrows = n_pad // NS
    zc = 80
    mesh = plsc.VectorSubcoreMesh(core_axis_name="c", subcore_axis_name="s")

    @functools.partial(
        pl.kernel,
        out_type=jax.ShapeDtypeStruct((NC, n_pad, dh), jnp.bfloat16),
        mesh=mesh,
        compiler_params=pltpu.CompilerParams(use_tc_tiling_on_sc=False),
        scratch_types=[
            pltpu.VMEM((kc, C), jnp.int32),
            pltpu.VMEM((kc, C), jnp.int32),
            pltpu.VMEM((C, dh), jnp.bfloat16),
            pltpu.VMEM((C, dh), jnp.bfloat16),
            pltpu.VMEM((zc, dh), jnp.bfloat16),
            pltpu.VMEM_SHARED((n_pad, dh), jnp.bfloat16),
            pltpu.SemaphoreType.DMA,
            pltpu.SemaphoreType.DMA,
            pltpu.SemaphoreType.DMA,
            pltpu.SemaphoreType.DMA,
        ],
    )
    def scatter_kernel(h2a_hbm, h2b_hbm, src_hbm, dst_hbm, outp_hbm,
                       src_v, dst_v, rows_a, rows_b, zrows_v, out_sh,
                       gs_a, gs_b, ss_a, ss_b):
        c = lax.axis_index("c")
        s = lax.axis_index("s")
        pltpu.sync_copy(src_hbm.at[s], src_v)
        pltpu.sync_copy(dst_hbm.at[s], dst_v)

        def zfill(r, _):
            for k in range(dh // 32):
                zrows_v[r, pl.ds(k * 32, 32)] = jnp.zeros((32,), jnp.bfloat16)
            return 0

        lax.fori_loop(0, zc, zfill, 0)
        base = s * zrows
        for t in range(zrows // zc):
            pltpu.sync_copy(zrows_v, out_sh.at[pl.ds(base + t * zc, zc)])
        plsc.subcore_barrier()

        def edge_loop(h2_hbm):
            pltpu.async_copy(h2_hbm.at[src_v.at[0]], rows_a, gs_a)

            def step(g, _):
                j = 2 * g
                pltpu.make_async_copy(
                    h2_hbm.at[src_v.at[j]], rows_a, gs_a).wait()

                @pl.when(g > 0)
                def _():
                    pltpu.make_async_copy(
                        rows_b, out_sh.at[dst_v.at[j - 1]], ss_b).wait()

                gb = pltpu.async_copy(h2_hbm.at[src_v.at[j + 1]], rows_b, gs_b)
                sa = pltpu.async_copy(rows_a, out_sh.at[dst_v.at[j]], ss_a,
                                      add=True)
                gb.wait()
                sa.wait()

                @pl.when(j + 2 < kc)
                def _():
                    pltpu.async_copy(h2_hbm.at[src_v.at[j + 2]], rows_a, gs_a)

                pltpu.async_copy(rows_b, out_sh.at[dst_v.at[j + 1]], ss_b,
                                 add=True)
                return 0

            lax.fori_loop(0, kc // 2, step, 0)
            pltpu.make_async_copy(
                rows_b, out_sh.at[dst_v.at[kc - 1]], ss_b).wait()

        @pl.when(c == 0)
        def _():
            edge_loop(h2a_hbm)

        @pl.when(c == 1)
        def _():
            edge_loop(h2b_hbm)

        plsc.subcore_barrier()
        pltpu.sync_copy(out_sh.at[pl.ds(base, zrows)],
                        outp_hbm.at[c, pl.ds(base, zrows)])

    return scatter_kernel


def _h2_body(x_ref, w_ref, b_ref, degp_ref, h2a_ref, h2b_ref, dinv_ref):
    deg = jnp.sum(degp_ref[...], axis=1, keepdims=True) + 1.0
    dinv = lax.rsqrt(deg)
    h = jnp.dot(x_ref[...], w_ref[...], preferred_element_type=jnp.float32)
    h2 = (h + b_ref[...]) * dinv
    dh = h2.shape[1] // 2
    h2a_ref[...] = h2[:, :dh].astype(jnp.bfloat16)
    h2b_ref[...] = h2[:, dh:].astype(jnp.bfloat16)
    dinv_ref[...] = dinv


def _final_body(outp_ref, h2a_ref, h2b_ref, dinv_ref, out_ref):
    dinv = dinv_ref[...]
    dh = h2a_ref.shape[1]
    for i, h2r in enumerate([h2a_ref, h2b_ref]):
        acc = outp_ref[i, :, :].astype(jnp.float32)
        out_ref[:, i * dh:(i + 1) * dh] = jnp.maximum(
            dinv * (acc + h2r[...].astype(jnp.float32)), 0.0)


def kernel(x, edge_index, W, b):
    n, d = x.shape
    e = edge_index.shape[1]
    dh = d // NC

    n_pad = ((n + NS * 16 - 1) // (NS * 16)) * (NS * 16)
    e_pad = ((e + NW * C - 1) // (NW * C)) * (NW * C)
    kc = e_pad // (NS * C)
    kcd = e_pad // (NW * C)

    src = edge_index[0]
    dst = edge_index[1]
    if e_pad != e:
        src = jnp.concatenate([src, jnp.zeros((e_pad - e,), jnp.int32)])
        dst = jnp.concatenate(
            [dst, jnp.full((e_pad - e,), n_pad - 1, jnp.int32)])
    src_r = src.reshape(NS, kc, C)
    dst_r = dst.reshape(NS, kc, C)
    dst_rd = dst.reshape(NW, kcd, C)

    x_pad = jnp.pad(x, ((0, n_pad - n), (0, 0))) if n_pad != n else x

    degp = _deg_kernel(n_pad, kcd)(dst_rd)
    degp_t = degp.T

    h2a, h2b, dinv = pl.pallas_call(
        _h2_body,
        out_shape=[
            jax.ShapeDtypeStruct((n_pad, dh), jnp.bfloat16),
            jax.ShapeDtypeStruct((n_pad, dh), jnp.bfloat16),
            jax.ShapeDtypeStruct((n_pad, 1), jnp.float32),
        ],
    )(x_pad, W, b.reshape(1, d), degp_t)

    outp = _scatter_kernel(n_pad, dh, kc)(h2a, h2b, src_r, dst_r)

    out = pl.pallas_call(
        _final_body,
        out_shape=jax.ShapeDtypeStruct((n_pad, d), jnp.float32),
    )(outp, h2a, h2b, dinv)
    return out[:n]
